# baseline (speedup 1.0000x reference)
import functools

import jax
import jax.numpy as jnp
from jax.experimental import pallas as pl

_N = 4096
_B = 4
_K = 32
_QB = 128
_NQB = _N // _QB


def _walk_body(has_noise, n_keys, xk_ref, xq_ref, c_ref, u_ref, *rest):
    if has_noise:
        nz_ref, out_ref = rest
    else:
        (out_ref,) = rest
    xk = xk_ref[0]
    xq = xq_ref[0]
    sqk = jnp.sum(xk * xk, axis=1)[None, :]
    sqq = jnp.sum(xq * xq, axis=1, keepdims=True)
    dot = jax.lax.dot_general(
        xq, xk, (((1,), (1,)), ((), ())),
        preferred_element_type=jnp.float32)
    d2 = sqq + sqk - 2.0 * dot

    iota = jax.lax.broadcasted_iota(
        jnp.int32, (_QB, n_keys), 1).astype(jnp.float32)
    c = c_ref[0]
    big = jnp.float32(n_keys)

    def _step(t, carry):
        masked, sel = carry
        m = jnp.min(masked, axis=1, keepdims=True)
        tie = jnp.where(masked == m, iota, big)
        amin = jnp.min(tie, axis=1, keepdims=True)
        sel = jnp.where(c == t, amin, sel)
        masked = jnp.where(tie == amin, jnp.float32(jnp.inf), masked)
        return masked, sel

    cmax = jnp.max(c)
    _, sel = jax.lax.fori_loop(
        0, cmax + 1, _step, (d2, jnp.zeros((_QB, 1), jnp.float32)))

    onehot = (iota == sel).astype(jnp.float32)
    nbr = jax.lax.dot_general(
        onehot, xk, (((1,), (0,)), ((), ())),
        precision=jax.lax.Precision.HIGHEST,
        preferred_element_type=jnp.float32)
    u = u_ref[0]
    new = xq + u * (nbr - xq)
    if has_noise:
        new = new + nz_ref[0]
    out_ref[0] = new


def _random_walk(x, noise, key):
    b, n, c_dim = x.shape
    k1, k2, k3 = jax.random.split(key, 3)
    choice = jax.random.randint(k1, (b, n, 1), 0, _K)
    u = jax.random.uniform(k2, (b, n, 1), dtype=x.dtype)
    nz = noise * jax.random.normal(k3, x.shape, dtype=x.dtype) if noise > 0.0 else None

    ch = choice[..., 0].astype(jnp.int32)
    perm = jnp.argsort(ch, axis=1, stable=True)
    xq_s = jnp.take_along_axis(x, perm[..., None], axis=1)
    c3 = jnp.take_along_axis(ch, perm, axis=1).reshape(b * _NQB, _QB, 1)
    u3 = jnp.take_along_axis(u[..., 0], perm, axis=1).reshape(b * _NQB, _QB, 1)
    in_specs = [
        pl.BlockSpec((1, n, c_dim), lambda bi, qi: (bi, 0, 0)),
        pl.BlockSpec((1, _QB, c_dim), lambda bi, qi: (bi, qi, 0)),
        pl.BlockSpec((1, _QB, 1), lambda bi, qi: (bi * _NQB + qi, 0, 0)),
        pl.BlockSpec((1, _QB, 1), lambda bi, qi: (bi * _NQB + qi, 0, 0)),
    ]
    args = [x, xq_s, c3, u3]
    if nz is not None:
        in_specs.append(pl.BlockSpec((1, _QB, c_dim), lambda bi, qi: (bi, qi, 0)))
        args.append(jnp.take_along_axis(nz, perm[..., None], axis=1))
    new = pl.pallas_call(
        functools.partial(_walk_body, nz is not None, n),
        grid=(b, _NQB),
        in_specs=in_specs,
        out_specs=pl.BlockSpec((1, _QB, c_dim), lambda bi, qi: (bi, qi, 0)),
        out_shape=jax.ShapeDtypeStruct((b, n, c_dim), jnp.float32),
    )(*args)
    inv = jnp.argsort(perm, axis=1)
    new = jnp.take_along_axis(new, inv[..., None], axis=1)
    return jnp.concatenate([x, new], axis=1)


def kernel(xyz, points):
    x1 = jnp.transpose(xyz[..., 0], (0, 2, 1))
    x2 = jnp.transpose(points[..., 0], (0, 2, 1))
    out1 = _random_walk(x1, 0.05, jax.random.key(1))
    out2 = _random_walk(x2, 0.0, jax.random.key(2))
    return (out1, out2)


# speedup vs baseline: 1.4163x; 1.4163x over previous
import functools

import jax
import jax.numpy as jnp
from jax.experimental import pallas as pl
from jax.experimental.pallas import tpu as pltpu

_N = 4096
_B = 4
_K = 32
_QB = 128
_NQB = _N // _QB


def _walk_body(has_noise, n_keys, xk_ref, xq_ref, c_ref, u_ref, *rest):
    if has_noise:
        nz_ref, out_ref, masked_ref, sel_ref = rest
    else:
        out_ref, masked_ref, sel_ref = rest
    xk = xk_ref[0]
    xq = xq_ref[0]
    sqk = jnp.sum(xk * xk, axis=1)[None, :]
    sqq = jnp.sum(xq * xq, axis=1, keepdims=True)
    dot = jax.lax.dot_general(
        xq, xk, (((1,), (1,)), ((), ())),
        preferred_element_type=jnp.float32)
    d2 = sqq + sqk - 2.0 * dot

    iota = jax.lax.broadcasted_iota(
        jnp.int32, (_QB, n_keys), 1).astype(jnp.float32)
    c = c_ref[0]
    big = jnp.float32(n_keys)

    cmax = jnp.max(c)
    masked_ref[...] = d2
    sel_ref[...] = jnp.zeros((_QB, 1), jnp.float32)
    for t in range(_K):
        @pl.when(t <= cmax)
        def _extract():
            masked = masked_ref[...]
            m = jnp.min(masked, axis=1, keepdims=True)
            tie = jnp.where(masked == m, iota, big)
            amin = jnp.min(tie, axis=1, keepdims=True)
            sel_ref[...] = jnp.where(c == t, amin, sel_ref[...])
            masked_ref[...] = jnp.where(tie == amin, jnp.float32(jnp.inf), masked)
    sel = sel_ref[...]

    onehot = (iota == sel).astype(jnp.float32)
    nbr = jax.lax.dot_general(
        onehot, xk, (((1,), (0,)), ((), ())),
        precision=jax.lax.Precision.HIGHEST,
        preferred_element_type=jnp.float32)
    u = u_ref[0]
    new = xq + u * (nbr - xq)
    if has_noise:
        new = new + nz_ref[0]
    out_ref[0] = new


def _random_walk(x, noise, key):
    b, n, c_dim = x.shape
    k1, k2, k3 = jax.random.split(key, 3)
    choice = jax.random.randint(k1, (b, n, 1), 0, _K)
    u = jax.random.uniform(k2, (b, n, 1), dtype=x.dtype)
    nz = noise * jax.random.normal(k3, x.shape, dtype=x.dtype) if noise > 0.0 else None

    ch = choice[..., 0].astype(jnp.int32)
    perm = jnp.argsort(ch, axis=1, stable=True)
    xq_s = jnp.take_along_axis(x, perm[..., None], axis=1)
    c3 = jnp.take_along_axis(ch, perm, axis=1).reshape(b * _NQB, _QB, 1)
    u3 = jnp.take_along_axis(u[..., 0], perm, axis=1).reshape(b * _NQB, _QB, 1)
    in_specs = [
        pl.BlockSpec((1, n, c_dim), lambda bi, qi: (bi, 0, 0)),
        pl.BlockSpec((1, _QB, c_dim), lambda bi, qi: (bi, qi, 0)),
        pl.BlockSpec((1, _QB, 1), lambda bi, qi: (bi * _NQB + qi, 0, 0)),
        pl.BlockSpec((1, _QB, 1), lambda bi, qi: (bi * _NQB + qi, 0, 0)),
    ]
    args = [x, xq_s, c3, u3]
    if nz is not None:
        in_specs.append(pl.BlockSpec((1, _QB, c_dim), lambda bi, qi: (bi, qi, 0)))
        args.append(jnp.take_along_axis(nz, perm[..., None], axis=1))
    new = pl.pallas_call(
        functools.partial(_walk_body, nz is not None, n),
        grid=(b, _NQB),
        in_specs=in_specs,
        out_specs=pl.BlockSpec((1, _QB, c_dim), lambda bi, qi: (bi, qi, 0)),
        out_shape=jax.ShapeDtypeStruct((b, n, c_dim), jnp.float32),
        scratch_shapes=[
            pltpu.VMEM((_QB, n), jnp.float32),
            pltpu.VMEM((_QB, 1), jnp.float32),
        ],
    )(*args)
    inv = jnp.argsort(perm, axis=1)
    new = jnp.take_along_axis(new, inv[..., None], axis=1)
    return jnp.concatenate([x, new], axis=1)


def kernel(xyz, points):
    x1 = jnp.transpose(xyz[..., 0], (0, 2, 1))
    x2 = jnp.transpose(points[..., 0], (0, 2, 1))
    out1 = _random_walk(x1, 0.05, jax.random.key(1))
    out2 = _random_walk(x2, 0.0, jax.random.key(2))
    return (out1, out2)


# speedup vs baseline: 1.5818x; 1.1168x over previous
import functools

import jax
import jax.numpy as jnp
from jax.experimental import pallas as pl
from jax.experimental.pallas import tpu as pltpu
from jax.experimental.pallas import tpu_sc as plsc

_N = 4096
_B = 4
_K = 32
_QB = 128
_NQB = _N // _QB


def _walk_body(mode, n_keys, xk_ref, xq_ref, c_ref, *rest):
    if mode == "noise":
        u_ref, nz_ref, out_ref, masked_ref, sel_ref = rest
    elif mode == "plain":
        u_ref, out_ref, masked_ref, sel_ref = rest
    else:
        out_ref, masked_ref, sel_ref = rest
    xk = xk_ref[0]
    xq = xq_ref[0]
    sqk = jnp.sum(xk * xk, axis=1)[None, :]
    sqq = jnp.sum(xq * xq, axis=1, keepdims=True)
    dot = jax.lax.dot_general(
        xq, xk, (((1,), (1,)), ((), ())),
        preferred_element_type=jnp.float32)
    d2 = sqq + sqk - 2.0 * dot

    iota = jax.lax.broadcasted_iota(
        jnp.int32, (_QB, n_keys), 1).astype(jnp.float32)
    c = c_ref[0]
    big = jnp.float32(n_keys)

    cmax = jnp.max(c)
    masked_ref[...] = d2
    sel_ref[...] = jnp.zeros((_QB, 1), jnp.float32)
    for t in range(_K):
        @pl.when(t <= cmax)
        def _extract():
            masked = masked_ref[...]
            m = jnp.min(masked, axis=1, keepdims=True)
            tie = jnp.where(masked == m, iota, big)
            amin = jnp.min(tie, axis=1, keepdims=True)
            sel_ref[...] = jnp.where(c == t, amin, sel_ref[...])
            masked_ref[...] = jnp.where(tie == amin, jnp.float32(jnp.inf), masked)
    sel = sel_ref[...]

    if mode == "sel":
        bi = pl.program_id(0)
        out_ref[0] = sel.astype(jnp.int32) + bi * n_keys
        return

    onehot = (iota == sel).astype(jnp.float32)
    nbr = jax.lax.dot_general(
        onehot, xk, (((1,), (0,)), ((), ())),
        precision=jax.lax.Precision.HIGHEST,
        preferred_element_type=jnp.float32)
    u = u_ref[0]
    new = xq + u * (nbr - xq)
    if mode == "noise":
        new = new + nz_ref[0]
    out_ref[0] = new


def _sc_gather_interp(x_flat, xqs_flat, gidx, u16):
    m_rows, d = x_flat.shape
    nw = 32
    per_w = m_rows // nw
    ch = 128
    mesh = plsc.VectorSubcoreMesh(core_axis_name="c", subcore_axis_name="s")

    @functools.partial(
        pl.kernel, mesh=mesh,
        out_type=jax.ShapeDtypeStruct((m_rows, d), jnp.float32),
        scratch_types=[
            pltpu.VMEM((ch,), jnp.int32),
            pltpu.VMEM((ch, d), jnp.float32),
            pltpu.VMEM((ch, d), jnp.float32),
            pltpu.VMEM((ch, 16), jnp.float32),
            pltpu.SemaphoreType.DMA,
        ],
    )
    def k(x_hbm, xq_hbm, gi_hbm, u_hbm, out_hbm, idx_v, nbr_v, xq_v, u_v, sem):
        wid = jax.lax.axis_index("s") * 2 + jax.lax.axis_index("c")

        @pl.loop(0, per_w // ch)
        def _chunk(ci):
            cb = wid * per_w + ci * ch
            pltpu.sync_copy(gi_hbm.at[pl.ds(cb, ch)], idx_v)
            pltpu.async_copy(x_hbm.at[idx_v], nbr_v, sem).wait()
            pltpu.sync_copy(xq_hbm.at[pl.ds(cb, ch)], xq_v)
            pltpu.sync_copy(u_hbm.at[pl.ds(cb, ch)], u_v)

            @pl.loop(0, ch)
            def _row(r):
                uu = u_v[r, :]
                for j in range(d // 16):
                    s = 16 * j
                    xqc = xq_v[r, pl.ds(s, 16)]
                    nbc = nbr_v[r, pl.ds(s, 16)]
                    nbr_v[r, pl.ds(s, 16)] = xqc + uu * (nbc - xqc)

            pltpu.sync_copy(nbr_v, out_hbm.at[pl.ds(cb, ch)])

    return k(x_flat, xqs_flat, gidx, u16)


def _random_walk(x, noise, key, use_sc=False):
    b, n, c_dim = x.shape
    k1, k2, k3 = jax.random.split(key, 3)
    choice = jax.random.randint(k1, (b, n, 1), 0, _K)
    u = jax.random.uniform(k2, (b, n, 1), dtype=x.dtype)
    nz = noise * jax.random.normal(k3, x.shape, dtype=x.dtype) if noise > 0.0 else None

    ch = choice[..., 0].astype(jnp.int32)
    perm = jnp.argsort(ch, axis=1, stable=True)
    xq_s = jnp.take_along_axis(x, perm[..., None], axis=1)
    c3 = jnp.take_along_axis(ch, perm, axis=1).reshape(b * _NQB, _QB, 1)
    u3 = jnp.take_along_axis(u[..., 0], perm, axis=1).reshape(b * _NQB, _QB, 1)
    in_specs = [
        pl.BlockSpec((1, n, c_dim), lambda bi, qi: (bi, 0, 0)),
        pl.BlockSpec((1, _QB, c_dim), lambda bi, qi: (bi, qi, 0)),
        pl.BlockSpec((1, _QB, 1), lambda bi, qi: (bi * _NQB + qi, 0, 0)),
    ]
    args = [x, xq_s, c3]
    scratch = [
        pltpu.VMEM((_QB, n), jnp.float32),
        pltpu.VMEM((_QB, 1), jnp.float32),
    ]
    if use_sc:
        sel_out = pl.pallas_call(
            functools.partial(_walk_body, "sel", n),
            grid=(b, _NQB),
            in_specs=in_specs,
            out_specs=pl.BlockSpec((1, _QB, 1), lambda bi, qi: (bi * _NQB + qi, 0, 0)),
            out_shape=jax.ShapeDtypeStruct((b * _NQB, _QB, 1), jnp.int32),
            scratch_shapes=scratch,
        )(*args)
        gidx = sel_out.reshape(b * n)
        u16 = jnp.tile(u3.reshape(b * n, 1), (1, 16))
        new = _sc_gather_interp(
            x.reshape(b * n, c_dim), xq_s.reshape(b * n, c_dim), gidx, u16)
        new = new.reshape(b, n, c_dim)
    else:
        mode = "noise" if nz is not None else "plain"
        in_specs.append(pl.BlockSpec((1, _QB, 1), lambda bi, qi: (bi * _NQB + qi, 0, 0)))
        args.append(u3)
        if nz is not None:
            in_specs.append(pl.BlockSpec((1, _QB, c_dim), lambda bi, qi: (bi, qi, 0)))
            args.append(jnp.take_along_axis(nz, perm[..., None], axis=1))
        new = pl.pallas_call(
            functools.partial(_walk_body, mode, n),
            grid=(b, _NQB),
            in_specs=in_specs,
            out_specs=pl.BlockSpec((1, _QB, c_dim), lambda bi, qi: (bi, qi, 0)),
            out_shape=jax.ShapeDtypeStruct((b, n, c_dim), jnp.float32),
            scratch_shapes=scratch,
        )(*args)
    inv = jnp.argsort(perm, axis=1)
    new = jnp.take_along_axis(new, inv[..., None], axis=1)
    return jnp.concatenate([x, new], axis=1)


def kernel(xyz, points):
    x1 = jnp.transpose(xyz[..., 0], (0, 2, 1))
    x2 = jnp.transpose(points[..., 0], (0, 2, 1))
    out1 = _random_walk(x1, 0.05, jax.random.key(1))
    out2 = _random_walk(x2, 0.0, jax.random.key(2), use_sc=True)
    return (out1, out2)


# speedup vs baseline: 1.5826x; 1.0005x over previous
import functools

import jax
import jax.numpy as jnp
from jax.experimental import pallas as pl
from jax.experimental.pallas import tpu as pltpu
from jax.experimental.pallas import tpu_sc as plsc

_N = 4096
_B = 4
_K = 32
_QB = 128
_NQB = _N // _QB


def _walk_body(mode, n_keys, xk_ref, xq_ref, c_ref, *rest):
    if mode == "noise":
        u_ref, nz_ref, out_ref, masked_ref, sel_ref = rest
    elif mode == "plain":
        u_ref, out_ref, masked_ref, sel_ref = rest
    else:
        out_ref, masked_ref, sel_ref = rest
    xk = xk_ref[0]
    xq = xq_ref[0]
    sqk = jnp.sum(xk * xk, axis=1)[None, :]
    sqq = jnp.sum(xq * xq, axis=1, keepdims=True)
    dot = jax.lax.dot_general(
        xq, xk, (((1,), (1,)), ((), ())),
        preferred_element_type=jnp.float32)
    d2 = sqq + sqk - 2.0 * dot

    iota = jax.lax.broadcasted_iota(
        jnp.int32, (_QB, n_keys), 1).astype(jnp.float32)
    c = c_ref[0]
    big = jnp.float32(n_keys)

    cmax = jnp.max(c)
    masked_ref[...] = d2
    sel_ref[...] = jnp.zeros((_QB, 1), jnp.float32)
    for t in range(_K):
        @pl.when(t <= cmax)
        def _extract():
            masked = masked_ref[...]
            m = jnp.min(masked, axis=1, keepdims=True)
            tie = jnp.where(masked == m, iota, big)
            amin = jnp.min(tie, axis=1, keepdims=True)
            sel_ref[...] = jnp.where(c == t, amin, sel_ref[...])
            masked_ref[...] = jnp.where(tie == amin, jnp.float32(jnp.inf), masked)
    sel = sel_ref[...]

    if mode == "sel":
        bi = pl.program_id(0)
        out_ref[0] = sel.astype(jnp.int32) + bi * n_keys
        return

    onehot = (iota == sel).astype(jnp.float32)
    nbr = jax.lax.dot_general(
        onehot, xk, (((1,), (0,)), ((), ())),
        precision=jax.lax.Precision.HIGHEST,
        preferred_element_type=jnp.float32)
    u = u_ref[0]
    new = xq + u * (nbr - xq)
    if mode == "noise":
        new = new + nz_ref[0]
    out_ref[0] = new


def _sc_gather_interp(x_flat, xqs_flat, gidx, u16):
    m_rows, d = x_flat.shape
    nw = 32
    per_w = m_rows // nw
    ch = 128
    mesh = plsc.VectorSubcoreMesh(core_axis_name="c", subcore_axis_name="s")

    @functools.partial(
        pl.kernel, mesh=mesh,
        out_type=jax.ShapeDtypeStruct((m_rows, d), jnp.float32),
        scratch_types=[
            pltpu.VMEM((ch,), jnp.int32),
            pltpu.VMEM((ch, d), jnp.float32),
            pltpu.VMEM((ch, d), jnp.float32),
            pltpu.VMEM((ch, 16), jnp.float32),
            pltpu.SemaphoreType.DMA,
        ],
    )
    def k(x_hbm, xq_hbm, gi_hbm, u_hbm, out_hbm, idx_v, nbr_v, xq_v, u_v, sem):
        wid = jax.lax.axis_index("s") * 2 + jax.lax.axis_index("c")

        @pl.loop(0, per_w // ch)
        def _chunk(ci):
            cb = wid * per_w + ci * ch
            pltpu.sync_copy(gi_hbm.at[pl.ds(cb, ch)], idx_v)
            pltpu.async_copy(x_hbm.at[idx_v], nbr_v, sem).wait()
            pltpu.sync_copy(xq_hbm.at[pl.ds(cb, ch)], xq_v)
            pltpu.sync_copy(u_hbm.at[pl.ds(cb, ch)], u_v)

            @pl.loop(0, ch)
            def _row(r):
                uu = u_v[r, :]
                for j in range(d // 16):
                    s = 16 * j
                    xqc = xq_v[r, pl.ds(s, 16)]
                    nbc = nbr_v[r, pl.ds(s, 16)]
                    nbr_v[r, pl.ds(s, 16)] = xqc + uu * (nbc - xqc)

            pltpu.sync_copy(nbr_v, out_hbm.at[pl.ds(cb, ch)])

    return k(x_flat, xqs_flat, gidx, u16)


def _random_walk(x, noise, key, use_sc=False):
    b, n, c_dim = x.shape
    k1, k2, k3 = jax.random.split(key, 3)
    choice = jax.random.randint(k1, (b, n, 1), 0, _K)
    u = jax.random.uniform(k2, (b, n, 1), dtype=x.dtype)
    nz = noise * jax.random.normal(k3, x.shape, dtype=x.dtype) if noise > 0.0 else None

    ch = choice[..., 0].astype(jnp.int32)
    perm = jnp.argsort(ch, axis=1, stable=True)
    xq_s = jnp.take_along_axis(x, perm[..., None], axis=1)
    c3 = jnp.take_along_axis(ch, perm, axis=1).reshape(b * _NQB, _QB, 1)
    u3 = jnp.take_along_axis(u[..., 0], perm, axis=1).reshape(b * _NQB, _QB, 1)
    in_specs = [
        pl.BlockSpec((1, n, c_dim), lambda bi, qi: (bi, 0, 0)),
        pl.BlockSpec((1, _QB, c_dim), lambda bi, qi: (bi, qi, 0)),
        pl.BlockSpec((1, _QB, 1), lambda bi, qi: (bi * _NQB + qi, 0, 0)),
    ]
    args = [x, xq_s, c3]
    scratch = [
        pltpu.VMEM((_QB, n), jnp.float32),
        pltpu.VMEM((_QB, 1), jnp.float32),
    ]
    if use_sc:
        sel_out = pl.pallas_call(
            functools.partial(_walk_body, "sel", n),
            grid=(b, _NQB),
            in_specs=in_specs,
            out_specs=pl.BlockSpec((1, _QB, 1), lambda bi, qi: (bi * _NQB + qi, 0, 0)),
            out_shape=jax.ShapeDtypeStruct((b * _NQB, _QB, 1), jnp.int32),
            scratch_shapes=scratch,
        )(*args)
        gidx = sel_out.reshape(b * n)
        u16 = jnp.tile(u3.reshape(b * n, 1), (1, 16))
        new = _sc_gather_interp(
            x.reshape(b * n, c_dim), xq_s.reshape(b * n, c_dim), gidx, u16)
        new = new.reshape(b, n, c_dim)
    else:
        mode = "noise" if nz is not None else "plain"
        in_specs.append(pl.BlockSpec((1, _QB, 1), lambda bi, qi: (bi * _NQB + qi, 0, 0)))
        args.append(u3)
        if nz is not None:
            in_specs.append(pl.BlockSpec((1, _QB, c_dim), lambda bi, qi: (bi, qi, 0)))
            args.append(jnp.take_along_axis(nz, perm[..., None], axis=1))
        new = pl.pallas_call(
            functools.partial(_walk_body, mode, n),
            grid=(b, _NQB),
            in_specs=in_specs,
            out_specs=pl.BlockSpec((1, _QB, c_dim), lambda bi, qi: (bi, qi, 0)),
            out_shape=jax.ShapeDtypeStruct((b, n, c_dim), jnp.float32),
            scratch_shapes=scratch,
        )(*args)
    inv = jnp.argsort(perm, axis=1)
    new = jnp.take_along_axis(new, inv[..., None], axis=1)
    return jnp.concatenate([x, new], axis=1)


def kernel(xyz, points):
    x1 = jnp.transpose(xyz[..., 0], (0, 2, 1))
    x2 = jnp.transpose(points[..., 0], (0, 2, 1))
    out2 = _random_walk(x2, 0.0, jax.random.key(2), use_sc=True)
    out1 = _random_walk(x1, 0.05, jax.random.key(1))
    return (out1, out2)
